# Initial kernel scaffold; baseline (speedup 1.0000x reference)
import jax
import jax.numpy as jnp
from jax.experimental import pallas as pl


def kernel(edge_attr, W0, W1, W2):
    raise NotImplementedError("write your pallas kernel here")


# speedup vs baseline: 4.7844x; 4.7844x over previous
import functools

import jax
import jax.numpy as jnp
from jax import lax
from jax.experimental import pallas as pl
from jax.experimental.pallas import tpu as pltpu
from jax.experimental.pallas import tpu_sc as plsc

EMB = 128
NTAB = 512
L = 16
NW = 32
C = 80


def _combine_tables_kernel(w0_ref, w1_ref, w2_ref, t_ref):
    row = lax.broadcasted_iota(jnp.int32, (NTAB, 16), 0)
    col = lax.broadcasted_iota(jnp.int32, (NTAB, 16), 1)
    oh0 = ((row & 7) == col).astype(jnp.float32)
    oh1 = (((row >> 3) & 7) == col).astype(jnp.float32)
    oh2 = ((row >> 6) == col).astype(jnp.float32)
    t_ref[...] = (
        jnp.dot(oh0, w0_ref[...], preferred_element_type=jnp.float32)
        + jnp.dot(oh1, w1_ref[...], preferred_element_type=jnp.float32)
        + jnp.dot(oh2, w2_ref[...], preferred_element_type=jnp.float32)
    )


def _combine_tables(W0, W1, W2):
    pads = [jnp.pad(w, ((0, 16 - w.shape[0]), (0, 0))) for w in (W0, W1, W2)]
    return pl.pallas_call(
        _combine_tables_kernel,
        out_shape=jax.ShapeDtypeStruct((NTAB, EMB), jnp.float32),
    )(*pads)


def _make_sc_gather(E):
    per_tile = E // NW
    n_chunks = per_tile // C
    mesh = plsc.VectorSubcoreMesh(core_axis_name="c", subcore_axis_name="s")

    @functools.partial(
        pl.kernel,
        mesh=mesh,
        out_type=jax.ShapeDtypeStruct((E, EMB), jnp.float32),
        scratch_types=[
            pltpu.VMEM((C,), jnp.int32),
            pltpu.VMEM((C,), jnp.int32),
            pltpu.VMEM((C,), jnp.int32),
            pltpu.VMEM((C,), jnp.int32),
            pltpu.VMEM((C, EMB), jnp.float32),
            pltpu.SemaphoreType.DMA,
        ],
    )
    def sc_gather(ea_hbm, table_hbm, out_hbm, e0_v, e1_v, e2_v, idx_v,
                  rows_v, sem):
        cid = lax.axis_index("c")
        sid = lax.axis_index("s")
        wid = sid * 2 + cid
        base = wid * per_tile

        def chunk_body(k, carry):
            cbase = base + k * C
            pltpu.sync_copy(ea_hbm.at[pl.ds(cbase, C)], e0_v)
            pltpu.sync_copy(ea_hbm.at[pl.ds(E + cbase, C)], e1_v)
            pltpu.sync_copy(ea_hbm.at[pl.ds(2 * E + cbase, C)], e2_v)
            for g in range(C // L):
                s = pl.ds(g * L, L)
                idx_v[s] = e0_v[s] + (e1_v[s] << 3) + (e2_v[s] << 6)
            pltpu.async_copy(table_hbm.at[idx_v], rows_v, sem).wait()
            pltpu.sync_copy(rows_v, out_hbm.at[pl.ds(cbase, C)])
            return carry

        lax.fori_loop(0, n_chunks, chunk_body, 0)

    return sc_gather


def kernel(edge_attr, W0, W1, W2):
    E = edge_attr.shape[0]
    table = _combine_tables(W0, W1, W2)
    ea = edge_attr.astype(jnp.int32).T.reshape(-1)
    return _make_sc_gather(E)(ea, table)


# speedup vs baseline: 7.6984x; 1.6091x over previous
import functools

import jax
import jax.numpy as jnp
from jax import lax
from jax.experimental import pallas as pl
from jax.experimental.pallas import tpu as pltpu
from jax.experimental.pallas import tpu_sc as plsc

EMB = 128
NTAB = 512
L = 16
NW = 32
C = 400
G = 80
EA_BLK = 2000


def _combine_tables_kernel(w0_ref, w1_ref, w2_ref, t_ref):
    row = lax.broadcasted_iota(jnp.int32, (NTAB, 16), 0)
    col = lax.broadcasted_iota(jnp.int32, (NTAB, 16), 1)
    oh0 = ((row & 7) == col).astype(jnp.float32)
    oh1 = (((row >> 3) & 7) == col).astype(jnp.float32)
    oh2 = ((row >> 6) == col).astype(jnp.float32)
    t_ref[...] = (
        jnp.dot(oh0, w0_ref[...], preferred_element_type=jnp.float32)
        + jnp.dot(oh1, w1_ref[...], preferred_element_type=jnp.float32)
        + jnp.dot(oh2, w2_ref[...], preferred_element_type=jnp.float32)
    )


def _combine_tables(W0, W1, W2):
    pads = [jnp.pad(w, ((0, 16 - w.shape[0]), (0, 0))) for w in (W0, W1, W2)]
    return pl.pallas_call(
        _combine_tables_kernel,
        out_shape=jax.ShapeDtypeStruct((NTAB, EMB), jnp.float32),
    )(*pads)


def _make_sc_gather(E):
    per_tile = E // NW
    n_chunks = per_tile // C
    n_ea = per_tile // EA_BLK
    mesh = plsc.VectorSubcoreMesh(core_axis_name="c", subcore_axis_name="s")

    @functools.partial(
        pl.kernel,
        mesh=mesh,
        out_type=jax.ShapeDtypeStruct((E, EMB), jnp.float32),
        scratch_types=[
            pltpu.VMEM((per_tile,), jnp.int32),
            pltpu.VMEM((EA_BLK,), jnp.int32),
            pltpu.VMEM((EA_BLK,), jnp.int32),
            pltpu.VMEM((EA_BLK,), jnp.int32),
            pltpu.VMEM((C, EMB), jnp.float32),
            pltpu.VMEM((C, EMB), jnp.float32),
            pltpu.SemaphoreType.DMA,
            pltpu.SemaphoreType.DMA,
            pltpu.SemaphoreType.DMA,
            pltpu.SemaphoreType.DMA,
        ],
    )
    def sc_gather(ea_hbm, table_hbm, out_hbm, idx_v, e0_v, e1_v, e2_v,
                  rows0, rows1, sg0, sg1, ss0, ss1):
        cid = lax.axis_index("c")
        sid = lax.axis_index("s")
        wid = sid * 2 + cid
        base = wid * per_tile

        def ea_round(r, carry):
            off = base + r * EA_BLK
            pltpu.sync_copy(ea_hbm.at[pl.ds(off, EA_BLK)], e0_v)
            pltpu.sync_copy(ea_hbm.at[pl.ds(E + off, EA_BLK)], e1_v)
            pltpu.sync_copy(ea_hbm.at[pl.ds(2 * E + off, EA_BLK)], e2_v)
            for g in range(EA_BLK // L):
                s = pl.ds(g * L, L)
                idx_v[pl.ds(r * EA_BLK + g * L, L)] = (
                    e0_v[s] + (e1_v[s] << 3) + (e2_v[s] << 6)
                )
            return carry

        lax.fori_loop(0, n_ea, ea_round, 0)

        def g_copy(c, rows, sem, j):
            return pltpu.make_async_copy(
                table_hbm.at[idx_v.at[pl.ds(c * C + j * G, G)]],
                rows.at[pl.ds(j * G, G)],
                sem,
            )

        def g_start(c, rows, sem):
            for j in range(C // G):
                g_copy(c, rows, sem, j).start()

        def g_wait(c, rows, sem):
            for j in range(C // G):
                g_copy(c, rows, sem, j).wait()

        def s_copy(c, rows, sem):
            return pltpu.make_async_copy(
                rows, out_hbm.at[pl.ds(base + c * C, C)], sem)

        g_start(0, rows0, sg0)
        g_start(1, rows1, sg1)

        def pipe_body(i, carry):
            c0 = 2 * i
            c1 = c0 + 1
            g_wait(c0, rows0, sg0)
            s_copy(c0, rows0, ss0).start()
            g_wait(c1, rows1, sg1)
            s_copy(c1, rows1, ss1).start()
            s_copy(c0, rows0, ss0).wait()
            g_start(c0 + 2, rows0, sg0)
            s_copy(c1, rows1, ss1).wait()
            g_start(c1 + 2, rows1, sg1)
            return carry

        half = (n_chunks - 3) // 2
        lax.fori_loop(0, half, pipe_body, 0)

        ca = n_chunks - 3
        cb = n_chunks - 2
        cc = n_chunks - 1
        g_wait(ca, rows0, sg0)
        s_copy(ca, rows0, ss0).start()
        g_wait(cb, rows1, sg1)
        s_copy(cb, rows1, ss1).start()
        s_copy(ca, rows0, ss0).wait()
        g_start(cc, rows0, sg0)
        g_wait(cc, rows0, sg0)
        s_copy(cb, rows1, ss1).wait()
        s_copy(cc, rows0, ss0).start()
        s_copy(cc, rows0, ss0).wait()

    return sc_gather


def kernel(edge_attr, W0, W1, W2):
    E = edge_attr.shape[0]
    table = _combine_tables(W0, W1, W2)
    ea = edge_attr.astype(jnp.int32).T.reshape(-1)
    return _make_sc_gather(E)(ea, table)


# speedup vs baseline: 14.6621x; 1.9046x over previous
import functools

import jax
import jax.numpy as jnp
from jax import lax
from jax.experimental import pallas as pl
from jax.experimental.pallas import tpu as pltpu
from jax.experimental.pallas import tpu_sc as plsc

EMB = 128
NTAB = 512
L = 16
NW = 32
C = 400
G = 80
EA_BLK = 2000


def _combine_tables_kernel(w0_ref, w1_ref, w2_ref, t_ref):
    row = lax.broadcasted_iota(jnp.int32, (NTAB, 16), 0)
    col = lax.broadcasted_iota(jnp.int32, (NTAB, 16), 1)
    oh0 = ((row & 7) == col).astype(jnp.float32)
    oh1 = (((row >> 3) & 7) == col).astype(jnp.float32)
    oh2 = ((row >> 6) == col).astype(jnp.float32)
    t_ref[...] = (
        jnp.dot(oh0, w0_ref[...], preferred_element_type=jnp.float32)
        + jnp.dot(oh1, w1_ref[...], preferred_element_type=jnp.float32)
        + jnp.dot(oh2, w2_ref[...], preferred_element_type=jnp.float32)
    )


def _combine_tables(W0, W1, W2):
    pads = [jnp.pad(w, ((0, 16 - w.shape[0]), (0, 0))) for w in (W0, W1, W2)]
    return pl.pallas_call(
        _combine_tables_kernel,
        out_shape=jax.ShapeDtypeStruct((NTAB, EMB), jnp.float32),
    )(*pads)


def _make_sc_gather(E):
    per_tile = E // NW
    n_chunks = per_tile // C
    n_ea = per_tile // EA_BLK
    mesh = plsc.VectorSubcoreMesh(core_axis_name="c", subcore_axis_name="s")

    @functools.partial(
        pl.kernel,
        mesh=mesh,
        out_type=jax.ShapeDtypeStruct((E, EMB), jnp.float32),
        scratch_types=[
            pltpu.VMEM((per_tile,), jnp.int32),
            pltpu.VMEM((EA_BLK,), jnp.int32),
            pltpu.VMEM((EA_BLK,), jnp.int32),
            pltpu.VMEM((EA_BLK,), jnp.int32),
            pltpu.VMEM((C, EMB), jnp.float32),
            pltpu.VMEM((C, EMB), jnp.float32),
            pltpu.VMEM_SHARED((NTAB, EMB), jnp.float32),
            pltpu.SemaphoreType.DMA,
            pltpu.SemaphoreType.DMA,
            pltpu.SemaphoreType.DMA,
            pltpu.SemaphoreType.DMA,
        ],
    )
    def sc_gather(ea_hbm, table_hbm, out_hbm, idx_v, e0_v, e1_v, e2_v,
                  rows0, rows1, table_sh, sg0, sg1, ss0, ss1):
        cid = lax.axis_index("c")
        sid = lax.axis_index("s")
        wid = sid * 2 + cid
        base = wid * per_tile

        @pl.when(sid == 0)
        def _():
            pltpu.sync_copy(table_hbm, table_sh)

        def ea_round(r, carry):
            off = base + r * EA_BLK
            pltpu.sync_copy(ea_hbm.at[pl.ds(off, EA_BLK)], e0_v)
            pltpu.sync_copy(ea_hbm.at[pl.ds(E + off, EA_BLK)], e1_v)
            pltpu.sync_copy(ea_hbm.at[pl.ds(2 * E + off, EA_BLK)], e2_v)
            for g in range(EA_BLK // L):
                s = pl.ds(g * L, L)
                idx_v[pl.ds(r * EA_BLK + g * L, L)] = (
                    e0_v[s] + (e1_v[s] << 3) + (e2_v[s] << 6)
                )
            return carry

        lax.fori_loop(0, n_ea, ea_round, 0)

        plsc.subcore_barrier()

        def g_copy(c, rows, sem, j):
            return pltpu.make_async_copy(
                table_sh.at[idx_v.at[pl.ds(c * C + j * G, G)]],
                rows.at[pl.ds(j * G, G)],
                sem,
            )

        def g_start(c, rows, sem):
            for j in range(C // G):
                g_copy(c, rows, sem, j).start()

        def g_wait(c, rows, sem):
            for j in range(C // G):
                g_copy(c, rows, sem, j).wait()

        def s_copy(c, rows, sem):
            return pltpu.make_async_copy(
                rows, out_hbm.at[pl.ds(base + c * C, C)], sem)

        g_start(0, rows0, sg0)
        g_start(1, rows1, sg1)

        def pipe_body(i, carry):
            c0 = 2 * i
            c1 = c0 + 1
            g_wait(c0, rows0, sg0)
            s_copy(c0, rows0, ss0).start()
            g_wait(c1, rows1, sg1)
            s_copy(c1, rows1, ss1).start()
            s_copy(c0, rows0, ss0).wait()
            g_start(c0 + 2, rows0, sg0)
            s_copy(c1, rows1, ss1).wait()
            g_start(c1 + 2, rows1, sg1)
            return carry

        half = (n_chunks - 3) // 2
        lax.fori_loop(0, half, pipe_body, 0)

        ca = n_chunks - 3
        cb = n_chunks - 2
        cc = n_chunks - 1
        g_wait(ca, rows0, sg0)
        s_copy(ca, rows0, ss0).start()
        g_wait(cb, rows1, sg1)
        s_copy(cb, rows1, ss1).start()
        s_copy(ca, rows0, ss0).wait()
        g_start(cc, rows0, sg0)
        g_wait(cc, rows0, sg0)
        s_copy(cb, rows1, ss1).wait()
        s_copy(cc, rows0, ss0).start()
        s_copy(cc, rows0, ss0).wait()

    return sc_gather


def kernel(edge_attr, W0, W1, W2):
    E = edge_attr.shape[0]
    table = _combine_tables(W0, W1, W2)
    ea = edge_attr.astype(jnp.int32).T.reshape(-1)
    return _make_sc_gather(E)(ea, table)


# speedup vs baseline: 19.0433x; 1.2988x over previous
import functools

import jax
import jax.numpy as jnp
from jax import lax
from jax.experimental import pallas as pl
from jax.experimental.pallas import tpu as pltpu
from jax.experimental.pallas import tpu_sc as plsc

EMB = 128
NTAB = 512
L = 16
NW = 32
C = 80
NBUF = 5
EA_BLK = 2000


def _combine_tables_kernel(w0_ref, w1_ref, w2_ref, t_ref):
    row = lax.broadcasted_iota(jnp.int32, (NTAB, 16), 0)
    col = lax.broadcasted_iota(jnp.int32, (NTAB, 16), 1)
    oh0 = ((row & 7) == col).astype(jnp.float32)
    oh1 = (((row >> 3) & 7) == col).astype(jnp.float32)
    oh2 = ((row >> 6) == col).astype(jnp.float32)
    t_ref[...] = (
        jnp.dot(oh0, w0_ref[...], preferred_element_type=jnp.float32)
        + jnp.dot(oh1, w1_ref[...], preferred_element_type=jnp.float32)
        + jnp.dot(oh2, w2_ref[...], preferred_element_type=jnp.float32)
    )


def _combine_tables(W0, W1, W2):
    pads = [jnp.pad(w, ((0, 16 - w.shape[0]), (0, 0))) for w in (W0, W1, W2)]
    return pl.pallas_call(
        _combine_tables_kernel,
        out_shape=jax.ShapeDtypeStruct((NTAB, EMB), jnp.float32),
    )(*pads)


def _make_sc_gather(E):
    per_tile = E // NW
    n_chunks = per_tile // C
    n_ea = per_tile // EA_BLK
    mesh = plsc.VectorSubcoreMesh(core_axis_name="c", subcore_axis_name="s")

    @functools.partial(
        pl.kernel,
        mesh=mesh,
        out_type=jax.ShapeDtypeStruct((E, EMB), jnp.float32),
        scratch_types=[
            pltpu.VMEM((per_tile,), jnp.int32),
            pltpu.VMEM((EA_BLK,), jnp.int32),
            pltpu.VMEM((EA_BLK,), jnp.int32),
            pltpu.VMEM((EA_BLK,), jnp.int32),
        ] + [pltpu.VMEM((C, EMB), jnp.float32) for _ in range(NBUF)]
        + [pltpu.VMEM_SHARED((NTAB, EMB), jnp.float32)]
        + [pltpu.SemaphoreType.DMA for _ in range(2 * NBUF)],
    )
    def sc_gather(ea_hbm, table_hbm, out_hbm, idx_v, e0_v, e1_v, e2_v,
                  *bufs_and_sems):
        rows = bufs_and_sems[:NBUF]
        table_sh = bufs_and_sems[NBUF]
        sg = bufs_and_sems[NBUF + 1:2 * NBUF + 1]
        ss = bufs_and_sems[2 * NBUF + 1:]
        cid = lax.axis_index("c")
        sid = lax.axis_index("s")
        wid = sid * 2 + cid
        base = wid * per_tile

        @pl.when(sid == 0)
        def _():
            pltpu.sync_copy(table_hbm, table_sh)

        def ea_round(r, carry):
            off = base + r * EA_BLK
            pltpu.sync_copy(ea_hbm.at[pl.ds(off, EA_BLK)], e0_v)
            pltpu.sync_copy(ea_hbm.at[pl.ds(E + off, EA_BLK)], e1_v)
            pltpu.sync_copy(ea_hbm.at[pl.ds(2 * E + off, EA_BLK)], e2_v)
            for g in range(EA_BLK // L):
                s = pl.ds(g * L, L)
                idx_v[pl.ds(r * EA_BLK + g * L, L)] = (
                    e0_v[s] + (e1_v[s] << 3) + (e2_v[s] << 6)
                )
            return carry

        lax.fori_loop(0, n_ea, ea_round, 0)

        plsc.subcore_barrier()

        def g_copy(c, b):
            return pltpu.make_async_copy(
                table_sh.at[idx_v.at[pl.ds(c * C, C)]], rows[b], sg[b])

        def s_copy(c, b):
            return pltpu.make_async_copy(
                rows[b], out_hbm.at[pl.ds(base + c * C, C)], ss[b])

        n_groups = n_chunks // NBUF
        for b in range(NBUF):
            g_copy(b, b).start()

        def pipe_group(i, carry):
            c0 = i * NBUF
            for b in range(NBUF):
                g_copy(c0 + b, b).wait()
                s_copy(c0 + b, b).start()
            for b in range(NBUF):
                s_copy(c0 + b, b).wait()
                g_copy(c0 + NBUF + b, b).start()
            return carry

        lax.fori_loop(0, n_groups - 1, pipe_group, 0)

        cl = (n_groups - 1) * NBUF
        for b in range(NBUF):
            g_copy(cl + b, b).wait()
            s_copy(cl + b, b).start()
        for b in range(NBUF):
            s_copy(cl + b, b).wait()

    return sc_gather


def kernel(edge_attr, W0, W1, W2):
    E = edge_attr.shape[0]
    table = _combine_tables(W0, W1, W2)
    ea = edge_attr.astype(jnp.int32).T.reshape(-1)
    return _make_sc_gather(E)(ea, table)
